# Initial kernel scaffold; baseline (speedup 1.0000x reference)
import jax
import jax.numpy as jnp
from jax.experimental import pallas as pl


def kernel(embs_local_global, cu_seqlens, Wq, Wk, Wv, bq, bk, bv):
    raise NotImplementedError("write your pallas kernel here")


# speedup vs baseline: 4.4479x; 4.4479x over previous
import functools

import jax
import jax.numpy as jnp
from jax.experimental import pallas as pl
from jax.experimental.pallas import tpu as pltpu

_L = 512


def _seg_attn_kernel(cu_ref, x_ref, w_ref, b_ref, out_ref):
    b = pl.program_id(0)
    start = cu_ref[b]
    n = cu_ref[b + 1] - start

    x = x_ref[pl.ds(start, _L), :]
    qkv = jax.lax.dot_general(
        x, w_ref[:, :], (((1,), (0,)), ((), ())),
        preferred_element_type=jnp.float32,
        precision=jax.lax.Precision.HIGHEST,
    ) + b_ref[0, :]
    d = x_ref.shape[1]
    q = qkv[:, :d]
    k = qkv[:, d:2 * d]
    v = qkv[:, 2 * d:]

    s = jax.lax.dot_general(
        q, k, (((1,), (1,)), ((), ())),
        preferred_element_type=jnp.float32,
        precision=jax.lax.Precision.HIGHEST,
    )
    ii = jax.lax.broadcasted_iota(jnp.int32, (_L, _L), 0)
    jj = jax.lax.broadcasted_iota(jnp.int32, (_L, _L), 1)
    valid = (jj < n) & (jj != ii)
    s = jnp.where(valid, s, -jnp.inf)
    m = jnp.max(s, axis=1, keepdims=True)
    m = jnp.maximum(m, jnp.float32(-1e30))
    p = jnp.exp(s - m)
    denom = jnp.sum(p, axis=1, keepdims=True)
    attn = p / jnp.maximum(denom, jnp.float32(1e-30))
    o = jax.lax.dot_general(
        attn, v, (((1,), (0,)), ((), ())),
        preferred_element_type=jnp.float32,
        precision=jax.lax.Precision.HIGHEST,
    )
    out_ref[pl.ds(start, _L), :] = o


@functools.partial(jax.jit, static_argnames=())
def kernel(embs_local_global, cu_seqlens, Wq, Wk, Wv, bq, bk, bv):
    t, d = embs_local_global.shape
    b_count = cu_seqlens.shape[0] - 1
    x_pad = jnp.concatenate(
        [embs_local_global, jnp.zeros((_L, d), embs_local_global.dtype)], axis=0)
    w = jnp.concatenate([Wq, Wk, Wv], axis=1)
    bias = jnp.concatenate([bq, bk, bv])[None, :]

    grid_spec = pltpu.PrefetchScalarGridSpec(
        num_scalar_prefetch=1,
        grid=(b_count,),
        in_specs=[
            pl.BlockSpec((t + _L, d), lambda b, cu: (0, 0)),
            pl.BlockSpec((d, 3 * d), lambda b, cu: (0, 0)),
            pl.BlockSpec((1, 3 * d), lambda b, cu: (0, 0)),
        ],
        out_specs=pl.BlockSpec((t + _L, d), lambda b, cu: (0, 0)),
    )
    out = pl.pallas_call(
        _seg_attn_kernel,
        grid_spec=grid_spec,
        out_shape=jax.ShapeDtypeStruct((t + _L, d), jnp.float32),
        compiler_params=pltpu.CompilerParams(
            dimension_semantics=("arbitrary",),
        ),
    )(cu_seqlens, x_pad, w, bias)
    return out[:t]


# speedup vs baseline: 11.4715x; 2.5791x over previous
import functools

import jax
import jax.numpy as jnp
from jax.experimental import pallas as pl
from jax.experimental.pallas import tpu as pltpu

_L = 512


def _seg_attn_kernel(cu_ref, x_ref, w_ref, b_ref, out_ref):
    b = pl.program_id(0)
    start = cu_ref[b]
    n = cu_ref[b + 1] - start

    x = x_ref[pl.ds(start, _L), :]
    qkv = jax.lax.dot_general(
        x, w_ref[:, :], (((1,), (0,)), ((), ())),
        preferred_element_type=jnp.float32,
        precision=jax.lax.Precision.DEFAULT,
    ) + b_ref[0, :]
    d = x_ref.shape[1]
    q = qkv[:, :d]
    k = qkv[:, d:2 * d]
    v = qkv[:, 2 * d:]

    s = jax.lax.dot_general(
        q, k, (((1,), (1,)), ((), ())),
        preferred_element_type=jnp.float32,
        precision=jax.lax.Precision.DEFAULT,
    )
    ii = jax.lax.broadcasted_iota(jnp.int32, (_L, _L), 0)
    jj = jax.lax.broadcasted_iota(jnp.int32, (_L, _L), 1)
    valid = (jj < n) & (jj != ii)
    s = jnp.where(valid, s, -jnp.inf)
    m = jnp.max(s, axis=1, keepdims=True)
    m = jnp.maximum(m, jnp.float32(-1e30))
    p = jnp.exp(s - m)
    denom = jnp.sum(p, axis=1, keepdims=True)
    attn = p / jnp.maximum(denom, jnp.float32(1e-30))
    o = jax.lax.dot_general(
        attn, v, (((1,), (0,)), ((), ())),
        preferred_element_type=jnp.float32,
        precision=jax.lax.Precision.DEFAULT,
    )
    out_ref[pl.ds(start, _L), :] = o


@functools.partial(jax.jit, static_argnames=())
def kernel(embs_local_global, cu_seqlens, Wq, Wk, Wv, bq, bk, bv):
    t, d = embs_local_global.shape
    b_count = cu_seqlens.shape[0] - 1
    x_pad = jnp.concatenate(
        [embs_local_global, jnp.zeros((_L, d), embs_local_global.dtype)], axis=0)
    w = jnp.concatenate([Wq, Wk, Wv], axis=1)
    bias = jnp.concatenate([bq, bk, bv])[None, :]

    grid_spec = pltpu.PrefetchScalarGridSpec(
        num_scalar_prefetch=1,
        grid=(b_count,),
        in_specs=[
            pl.BlockSpec((t + _L, d), lambda b, cu: (0, 0)),
            pl.BlockSpec((d, 3 * d), lambda b, cu: (0, 0)),
            pl.BlockSpec((1, 3 * d), lambda b, cu: (0, 0)),
        ],
        out_specs=pl.BlockSpec((t + _L, d), lambda b, cu: (0, 0)),
    )
    out = pl.pallas_call(
        _seg_attn_kernel,
        grid_spec=grid_spec,
        out_shape=jax.ShapeDtypeStruct((t + _L, d), jnp.float32),
        compiler_params=pltpu.CompilerParams(
            dimension_semantics=("arbitrary",),
        ),
    )(cu_seqlens, x_pad, w, bias)
    return out[:t]


# speedup vs baseline: 14.9243x; 1.3010x over previous
import functools

import jax
import jax.numpy as jnp
from jax.experimental import pallas as pl
from jax.experimental.pallas import tpu as pltpu

_L = 512


def _seg_attn_kernel(cu_ref, x_ref, w_ref, b_ref, out_ref, q_ref, k_ref, v_ref):
    b = pl.program_id(0)
    d = x_ref.shape[1]

    @pl.when(b == 0)
    def _project():
        qkv = jax.lax.dot_general(
            x_ref[...], w_ref[...], (((1,), (0,)), ((), ())),
            preferred_element_type=jnp.float32,
        ) + b_ref[0, :]
        q_ref[...] = qkv[:, :d]
        k_ref[...] = qkv[:, d:2 * d]
        v_ref[...] = qkv[:, 2 * d:]

    @pl.when(b > 0)
    def _attend():
        for sub in range(2):
            seg = 2 * (b - 1) + sub
            start = cu_ref[seg]
            n = cu_ref[seg + 1] - start
            q = q_ref[pl.ds(start, _L), :]
            k = k_ref[pl.ds(start, _L), :]
            v = v_ref[pl.ds(start, _L), :]
            s = jax.lax.dot_general(
                q, k, (((1,), (1,)), ((), ())),
                preferred_element_type=jnp.float32,
            )
            ii = jax.lax.broadcasted_iota(jnp.int32, (_L, _L), 0)
            jj = jax.lax.broadcasted_iota(jnp.int32, (_L, _L), 1)
            valid = (jj < n) & (jj != ii)
            s = jnp.where(valid, s, -jnp.inf)
            m = jnp.max(s, axis=1, keepdims=True)
            m = jnp.maximum(m, jnp.float32(-1e30))
            p = jnp.exp(s - m)
            denom = jnp.sum(p, axis=1, keepdims=True)
            attn = p / jnp.maximum(denom, jnp.float32(1e-30))
            o = jax.lax.dot_general(
                attn, v, (((1,), (0,)), ((), ())),
                preferred_element_type=jnp.float32,
            )
            out_ref[pl.ds(start, _L), :] = o


@functools.partial(jax.jit, static_argnames=())
def kernel(embs_local_global, cu_seqlens, Wq, Wk, Wv, bq, bk, bv):
    t, d = embs_local_global.shape
    b_count = cu_seqlens.shape[0] - 1
    x_pad = jnp.concatenate(
        [embs_local_global, jnp.zeros((_L, d), embs_local_global.dtype)], axis=0)
    w = jnp.concatenate([Wq, Wk, Wv], axis=1)
    bias = jnp.concatenate([bq, bk, bv])[None, :]

    grid_spec = pltpu.PrefetchScalarGridSpec(
        num_scalar_prefetch=1,
        grid=(1 + b_count // 2,),
        in_specs=[
            pl.BlockSpec((t + _L, d), lambda b, cu: (0, 0)),
            pl.BlockSpec((d, 3 * d), lambda b, cu: (0, 0)),
            pl.BlockSpec((1, 3 * d), lambda b, cu: (0, 0)),
        ],
        out_specs=pl.BlockSpec((t + _L, d), lambda b, cu: (0, 0)),
        scratch_shapes=[pltpu.VMEM((t + _L, d), jnp.float32)] * 3,
    )
    out = pl.pallas_call(
        _seg_attn_kernel,
        grid_spec=grid_spec,
        out_shape=jax.ShapeDtypeStruct((t + _L, d), jnp.float32),
        compiler_params=pltpu.CompilerParams(
            dimension_semantics=("arbitrary",),
        ),
    )(cu_seqlens, x_pad, w, bias)
    return out[:t]


# speedup vs baseline: 15.5008x; 1.0386x over previous
import functools

import jax
import jax.numpy as jnp
from jax.experimental import pallas as pl
from jax.experimental.pallas import tpu as pltpu

_L = 512
_NEG = -1e30


def _seg_attn_kernel(cu_ref, x_ref, w_ref, b_ref, out_ref,
                     q_ref, k_ref, v_ref, dpen_ref):
    b = pl.program_id(0)
    d = x_ref.shape[1]

    @pl.when(b == 0)
    def _project():
        qkv = jax.lax.dot_general(
            x_ref[...], w_ref[...], (((1,), (0,)), ((), ())),
            preferred_element_type=jnp.float32,
        ) + b_ref[0, :]
        q_ref[...] = qkv[:, :d]
        k_ref[...] = qkv[:, d:2 * d]
        v_ref[...] = qkv[:, 2 * d:]
        ii = jax.lax.broadcasted_iota(jnp.int32, (_L, _L), 0)
        jj = jax.lax.broadcasted_iota(jnp.int32, (_L, _L), 1)
        dpen_ref[...] = jnp.where(ii == jj, jnp.float32(_NEG), jnp.float32(0.0))

    @pl.when(b > 0)
    def _attend():
        for sub in range(2):
            seg = 2 * (b - 1) + sub
            start = cu_ref[seg]
            n = cu_ref[seg + 1] - start
            q = q_ref[pl.ds(start, _L), :]
            k = k_ref[pl.ds(start, _L), :]
            v = v_ref[pl.ds(start, _L), :]
            s = jax.lax.dot_general(
                q, k, (((1,), (1,)), ((), ())),
                preferred_element_type=jnp.float32,
            )
            jrow = jax.lax.broadcasted_iota(jnp.int32, (1, _L), 1)
            colpen = jnp.where(jrow < n, jnp.float32(0.0), jnp.float32(_NEG))
            s = s + dpen_ref[...] + colpen
            m = jnp.max(s, axis=1, keepdims=True)
            p = jnp.exp(s - m)
            denom = jnp.sum(p, axis=1, keepdims=True)
            o = jax.lax.dot_general(
                p, v, (((1,), (0,)), ((), ())),
                preferred_element_type=jnp.float32,
            ) / denom
            out_ref[pl.ds(start, _L), :] = o


@functools.partial(jax.jit, static_argnames=())
def kernel(embs_local_global, cu_seqlens, Wq, Wk, Wv, bq, bk, bv):
    t, d = embs_local_global.shape
    b_count = cu_seqlens.shape[0] - 1
    x_pad = jnp.concatenate(
        [embs_local_global, jnp.zeros((_L, d), embs_local_global.dtype)], axis=0)
    w = jnp.concatenate([Wq, Wk, Wv], axis=1)
    bias = jnp.concatenate([bq, bk, bv])[None, :]

    grid_spec = pltpu.PrefetchScalarGridSpec(
        num_scalar_prefetch=1,
        grid=(1 + b_count // 2,),
        in_specs=[
            pl.BlockSpec((t + _L, d), lambda b, cu: (0, 0)),
            pl.BlockSpec((d, 3 * d), lambda b, cu: (0, 0)),
            pl.BlockSpec((1, 3 * d), lambda b, cu: (0, 0)),
        ],
        out_specs=pl.BlockSpec((t + _L, d), lambda b, cu: (0, 0)),
        scratch_shapes=[pltpu.VMEM((t + _L, d), jnp.float32)] * 3
        + [pltpu.VMEM((_L, _L), jnp.float32)],
    )
    out = pl.pallas_call(
        _seg_attn_kernel,
        grid_spec=grid_spec,
        out_shape=jax.ShapeDtypeStruct((t + _L, d), jnp.float32),
        compiler_params=pltpu.CompilerParams(
            dimension_semantics=("arbitrary",),
        ),
    )(cu_seqlens, x_pad, w, bias)
    return out[:t]
